# Initial kernel scaffold; baseline (speedup 1.0000x reference)
import functools

import jax
import jax.numpy as jnp
from jax import lax
from jax.experimental import pallas as pl
from jax.experimental.pallas import tpu as pltpu

_K = 100
_PEAK_EPSILON = 1e-06


def _detgen_kernel(heat_ref, size_ref, off_ref,
                   sc_out_ref, box_out_ref, int_out_ref,
                   peaks_ref, *, H, W, C, K):
    WC = W * C
    x = heat_ref[0]
    p = jax.nn.sigmoid(x)

    neg_w = jnp.full((H, 1, C), -jnp.inf, dtype=jnp.float32)
    left = jnp.concatenate([neg_w, p[:, :-1, :]], axis=1)
    right = jnp.concatenate([p[:, 1:, :], neg_w], axis=1)
    mw = jnp.maximum(p, jnp.maximum(left, right))
    neg_h = jnp.full((1, W, C), -jnp.inf, dtype=jnp.float32)
    up = jnp.concatenate([neg_h, mw[:-1]], axis=0)
    dn = jnp.concatenate([mw[1:], neg_h], axis=0)
    m = jnp.maximum(mw, jnp.maximum(up, dn))

    peaks = jnp.where(jnp.abs(p - m) < _PEAK_EPSILON, p, 0.0)
    peaks_ref[...] = peaks

    rowmax = jnp.max(peaks, axis=(1, 2)).reshape(1, H)

    row_iota = lax.broadcasted_iota(jnp.int32, (1, H), 1)
    k_iota = lax.broadcasted_iota(jnp.int32, (1, H), 1)
    flat_pos = (lax.broadcasted_iota(jnp.int32, (1, W, C), 1) * C
                + lax.broadcasted_iota(jnp.int32, (1, W, C), 2))
    BIG = jnp.int32(1 << 30)

    def extract_body(i, state):
        rmax, sc_acc, id_acc = state
        gmax = jnp.max(rmax)
        y = jnp.min(jnp.where(rmax == gmax, row_iota, H))
        row = peaks_ref[pl.ds(y, 1)]
        pos = jnp.min(jnp.where(row == gmax, flat_pos, BIG))
        row2 = jnp.where(flat_pos == pos, -1.0, row)
        peaks_ref[pl.ds(y, 1)] = row2
        new_rmax = jnp.max(row2)
        rmax = jnp.where(row_iota == y, new_rmax, rmax)
        sc_acc = jnp.where(k_iota == i, gmax, sc_acc)
        id_acc = jnp.where(k_iota == i, y * WC + pos, id_acc)
        return rmax, sc_acc, id_acc

    sc0 = jnp.zeros((1, H), dtype=jnp.float32)
    id0 = jnp.zeros((1, H), dtype=jnp.int32)
    rowmax, sc, fid = lax.fori_loop(0, K, extract_body, (rowmax, sc0, id0))

    q = fid // C
    yv = q // W
    xv = q - yv * W
    cv = fid - q * C

    lane2 = lax.broadcasted_iota(jnp.int32, (1, 2 * W), 1)

    def gather_body(i, state):
        h_acc, w_acc, yo_acc, xo_acc = state
        sel = k_iota == i
        yi = jnp.sum(jnp.where(sel, yv, 0))
        xi = jnp.sum(jnp.where(sel, xv, 0))
        srow = size_ref[0, pl.ds(yi, 1), :]
        orow = off_ref[0, pl.ds(yi, 1), :]
        hs = jnp.sum(jnp.where(lane2 == 2 * xi, srow, 0.0))
        ws = jnp.sum(jnp.where(lane2 == 2 * xi + 1, srow, 0.0))
        yos = jnp.sum(jnp.where(lane2 == 2 * xi, orow, 0.0))
        xos = jnp.sum(jnp.where(lane2 == 2 * xi + 1, orow, 0.0))
        return (jnp.where(sel, hs, h_acc), jnp.where(sel, ws, w_acc),
                jnp.where(sel, yos, yo_acc), jnp.where(sel, xos, xo_acc))

    z = jnp.zeros((1, H), dtype=jnp.float32)
    h, w, yo, xo = lax.fori_loop(0, K, gather_body, (z, z, z, z))

    yf = yv.astype(jnp.float32)
    xf = xv.astype(jnp.float32)
    hh = jnp.maximum(h, 0.0)
    ww = jnp.maximum(w, 0.0)
    Hf = jnp.float32(H)
    Wf = jnp.float32(W)
    ymin = jnp.clip(yf + yo - hh / 2.0, 0.0, Hf)
    xmin = jnp.clip(xf + xo - ww / 2.0, 0.0, Wf)
    ymax = jnp.clip(yf + yo + hh / 2.0, 0.0, Hf)
    xmax = jnp.clip(xf + xo + ww / 2.0, 0.0, Wf)
    box = jnp.concatenate([ymin, xmin, ymax, xmax], axis=0)
    box = jnp.clip(box * 4.0 / 512.0, 0.0, 1.0)

    nd = jnp.sum(jnp.where((sc > 0.0) & (k_iota < K), 1, 0))
    nd_row = jnp.where(k_iota == 0, nd, 0)

    sc_out_ref[0] = sc[0]
    box_out_ref[0] = box
    int_out_ref[0] = jnp.concatenate([cv, nd_row], axis=0)


def kernel(ct_heatmaps, ct_size, ct_offset):
    B, H, W, C = ct_heatmaps.shape
    K = _K
    size_r = ct_size.reshape(B, H, 2 * W)
    off_r = ct_offset.reshape(B, H, 2 * W)

    body = functools.partial(_detgen_kernel, H=H, W=W, C=C, K=K)
    sc, box, ints = pl.pallas_call(
        body,
        grid=(B,),
        in_specs=[
            pl.BlockSpec((1, H, W, C), lambda b: (b, 0, 0, 0)),
            pl.BlockSpec((1, H, 2 * W), lambda b: (b, 0, 0)),
            pl.BlockSpec((1, H, 2 * W), lambda b: (b, 0, 0)),
        ],
        out_specs=[
            pl.BlockSpec((1, H), lambda b: (b, 0)),
            pl.BlockSpec((1, 4, H), lambda b: (b, 0, 0)),
            pl.BlockSpec((1, 2, H), lambda b: (b, 0, 0)),
        ],
        out_shape=[
            jax.ShapeDtypeStruct((B, H), jnp.float32),
            jax.ShapeDtypeStruct((B, 4, H), jnp.float32),
            jax.ShapeDtypeStruct((B, 2, H), jnp.int32),
        ],
        scratch_shapes=[pltpu.VMEM((H, W, C), jnp.float32)],
        compiler_params=pltpu.CompilerParams(
            dimension_semantics=("parallel",),
        ),
    )(ct_heatmaps, size_r, off_r)

    boxes = jnp.transpose(box, (0, 2, 1))[:, :K, :]
    channel_indices = ints[:, 0, :K]
    detection_scores = sc[:, :K]
    num_detections = ints[:, 1, 0]
    return boxes, channel_indices, detection_scores, num_detections


# speedup vs baseline: 5.0198x; 5.0198x over previous
import functools

import jax
import jax.numpy as jnp
from jax import lax
from jax.experimental import pallas as pl
from jax.experimental.pallas import tpu as pltpu

_K = 100
_PEAK_EPSILON = 1e-06


def _detgen_kernel(heat_ref, size_ref, off_ref,
                   sc_out_ref, box_out_ref, int_out_ref,
                   peaks_ref, *, H, W, C, K):
    WC = W * C
    x = heat_ref[0]
    p = jax.nn.sigmoid(x)

    neg_w = jnp.full((H, 1, C), -jnp.inf, dtype=jnp.float32)
    left = jnp.concatenate([neg_w, p[:, :-1, :]], axis=1)
    right = jnp.concatenate([p[:, 1:, :], neg_w], axis=1)
    mw = jnp.maximum(p, jnp.maximum(left, right))
    neg_h = jnp.full((1, W, C), -jnp.inf, dtype=jnp.float32)
    up = jnp.concatenate([neg_h, mw[:-1]], axis=0)
    dn = jnp.concatenate([mw[1:], neg_h], axis=0)
    m = jnp.maximum(mw, jnp.maximum(up, dn))

    peaks = jnp.where(jnp.abs(p - m) < _PEAK_EPSILON, p, 0.0)
    peaks_ref[...] = peaks

    rowmax = jnp.max(peaks, axis=(1, 2)).reshape(1, H)

    row_iota = lax.broadcasted_iota(jnp.int32, (1, H), 1)
    k_iota = lax.broadcasted_iota(jnp.int32, (1, H), 1)
    flat_pos = (lax.broadcasted_iota(jnp.int32, (1, W, C), 1) * C
                + lax.broadcasted_iota(jnp.int32, (1, W, C), 2))
    BIG = jnp.int32(1 << 30)

    def extract_body(i, state):
        rmax, sc_acc, id_acc = state
        gmax = jnp.max(rmax)
        y = jnp.min(jnp.where(rmax == gmax, row_iota, H))
        row = peaks_ref[pl.ds(y, 1)]
        pos = jnp.min(jnp.where(row == gmax, flat_pos, BIG))
        row2 = jnp.where(flat_pos == pos, -1.0, row)
        peaks_ref[pl.ds(y, 1)] = row2
        new_rmax = jnp.max(row2)
        rmax = jnp.where(row_iota == y, new_rmax, rmax)
        sc_acc = jnp.where(k_iota == i, gmax, sc_acc)
        id_acc = jnp.where(k_iota == i, y * WC + pos, id_acc)
        return rmax, sc_acc, id_acc

    sc0 = jnp.zeros((1, H), dtype=jnp.float32)
    id0 = jnp.zeros((1, H), dtype=jnp.int32)
    rowmax, sc, fid = lax.fori_loop(0, K, extract_body, (rowmax, sc0, id0))

    q = fid // C
    yv = q // W
    xv = q - yv * W
    cv = fid - q * C

    lane2 = lax.broadcasted_iota(jnp.int32, (1, 2 * W), 1)

    def gather_body(i, state):
        h_acc, w_acc, yo_acc, xo_acc = state
        sel = k_iota == i
        yi = jnp.sum(jnp.where(sel, yv, 0))
        xi = jnp.sum(jnp.where(sel, xv, 0))
        srow = size_ref[0, pl.ds(yi, 1), :]
        orow = off_ref[0, pl.ds(yi, 1), :]
        hs = jnp.sum(jnp.where(lane2 == 2 * xi, srow, 0.0))
        ws = jnp.sum(jnp.where(lane2 == 2 * xi + 1, srow, 0.0))
        yos = jnp.sum(jnp.where(lane2 == 2 * xi, orow, 0.0))
        xos = jnp.sum(jnp.where(lane2 == 2 * xi + 1, orow, 0.0))
        return (jnp.where(sel, hs, h_acc), jnp.where(sel, ws, w_acc),
                jnp.where(sel, yos, yo_acc), jnp.where(sel, xos, xo_acc))

    z = jnp.zeros((1, H), dtype=jnp.float32)
    h, w, yo, xo = lax.fori_loop(0, K, gather_body, (z, z, z, z))

    yf = yv.astype(jnp.float32)
    xf = xv.astype(jnp.float32)
    hh = jnp.maximum(h, 0.0)
    ww = jnp.maximum(w, 0.0)
    Hf = jnp.float32(H)
    Wf = jnp.float32(W)
    ymin = jnp.clip(yf + yo - hh / 2.0, 0.0, Hf)
    xmin = jnp.clip(xf + xo - ww / 2.0, 0.0, Wf)
    ymax = jnp.clip(yf + yo + hh / 2.0, 0.0, Hf)
    xmax = jnp.clip(xf + xo + ww / 2.0, 0.0, Wf)
    box = jnp.concatenate([ymin, xmin, ymax, xmax], axis=0)
    box = jnp.clip(box * 4.0 / 512.0, 0.0, 1.0)

    nd = jnp.sum(jnp.where((sc > 0.0) & (k_iota < K), 1, 0))
    nd_row = jnp.where(k_iota == 0, nd, 0)

    sc_out_ref[0, 0] = sc[0]
    box_out_ref[0] = box
    int_out_ref[0] = jnp.concatenate([cv, nd_row], axis=0)


def kernel(ct_heatmaps, ct_size, ct_offset):
    B, H, W, C = ct_heatmaps.shape
    K = _K
    size_r = ct_size.reshape(B, H, 2 * W)
    off_r = ct_offset.reshape(B, H, 2 * W)

    body = functools.partial(_detgen_kernel, H=H, W=W, C=C, K=K)
    sc, box, ints = pl.pallas_call(
        body,
        grid=(B,),
        in_specs=[
            pl.BlockSpec((1, H, W, C), lambda b: (b, 0, 0, 0)),
            pl.BlockSpec((1, H, 2 * W), lambda b: (b, 0, 0)),
            pl.BlockSpec((1, H, 2 * W), lambda b: (b, 0, 0)),
        ],
        out_specs=[
            pl.BlockSpec((1, 1, H), lambda b: (b, 0, 0)),
            pl.BlockSpec((1, 4, H), lambda b: (b, 0, 0)),
            pl.BlockSpec((1, 2, H), lambda b: (b, 0, 0)),
        ],
        out_shape=[
            jax.ShapeDtypeStruct((B, 1, H), jnp.float32),
            jax.ShapeDtypeStruct((B, 4, H), jnp.float32),
            jax.ShapeDtypeStruct((B, 2, H), jnp.int32),
        ],
        scratch_shapes=[pltpu.VMEM((H, W, C), jnp.float32)],
        compiler_params=pltpu.CompilerParams(
            dimension_semantics=("parallel",),
        ),
    )(ct_heatmaps, size_r, off_r)

    boxes = jnp.transpose(box, (0, 2, 1))[:, :K, :]
    channel_indices = ints[:, 0, :K]
    detection_scores = sc[:, 0, :K]
    num_detections = ints[:, 1, 0]
    return boxes, channel_indices, detection_scores, num_detections


# speedup vs baseline: 6.6635x; 1.3274x over previous
import functools

import jax
import jax.numpy as jnp
from jax import lax
from jax.experimental import pallas as pl
from jax.experimental.pallas import tpu as pltpu

_K = 100
_PEAK_EPSILON = 1e-06


def _detgen_kernel(heat_ref, size_ref, off_ref,
                   sc_out_ref, box_out_ref, int_out_ref,
                   peaks_ref, *, H, W, C, K):
    HW = H * W
    x = heat_ref[0]
    p = jax.nn.sigmoid(x)

    neg_w = jnp.full((H, 1, C), -jnp.inf, dtype=jnp.float32)
    left = jnp.concatenate([neg_w, p[:, :-1, :]], axis=1)
    right = jnp.concatenate([p[:, 1:, :], neg_w], axis=1)
    mw = jnp.maximum(p, jnp.maximum(left, right))
    neg_h = jnp.full((1, W, C), -jnp.inf, dtype=jnp.float32)
    up = jnp.concatenate([neg_h, mw[:-1]], axis=0)
    dn = jnp.concatenate([mw[1:], neg_h], axis=0)
    m = jnp.maximum(mw, jnp.maximum(up, dn))

    peaks = jnp.where(jnp.abs(p - m) < _PEAK_EPSILON, p, 0.0)
    peaks_ref[...] = peaks.reshape(HW // 8, 8, C)

    colmax = jnp.max(peaks, axis=2)

    pix_iota = (lax.broadcasted_iota(jnp.int32, (H, W), 0) * W
                + lax.broadcasted_iota(jnp.int32, (H, W), 1))
    k_iota = lax.broadcasted_iota(jnp.int32, (1, H), 1)
    s_iota = lax.broadcasted_iota(jnp.int32, (1, 8, C), 1)
    c_iota = lax.broadcasted_iota(jnp.int32, (1, 8, C), 2)
    BIG = jnp.int32(1 << 30)

    def extract_body(i, state):
        cmax, sc_acc, id_acc = state
        gmax = jnp.max(cmax)
        pix = jnp.min(jnp.where(cmax == gmax, pix_iota, BIG))
        g = pix // 8
        s = pix - g * 8
        page = peaks_ref[pl.ds(g, 1)]
        insub = s_iota == s
        c = jnp.min(jnp.where(insub & (page == gmax), c_iota, BIG))
        page2 = jnp.where(insub & (c_iota == c), -1.0, page)
        peaks_ref[pl.ds(g, 1)] = page2
        new_pixmax = jnp.max(jnp.where(insub, page2, -jnp.inf))
        cmax = jnp.where(pix_iota == pix, new_pixmax, cmax)
        sc_acc = jnp.where(k_iota == i, gmax, sc_acc)
        id_acc = jnp.where(k_iota == i, pix * C + c, id_acc)
        return cmax, sc_acc, id_acc

    sc0 = jnp.zeros((1, H), dtype=jnp.float32)
    id0 = jnp.zeros((1, H), dtype=jnp.int32)
    _, sc, fid = lax.fori_loop(0, K, extract_body, (colmax, sc0, id0))

    q = fid // C
    yv = q // W
    xv = q - yv * W
    cv = fid - q * C

    qT = q.reshape(H, 1)
    yT = qT // W
    xT = qT - yT * W
    lane_h = lax.broadcasted_iota(jnp.int32, (H, H), 1)
    onehot = (yT == lane_h).astype(jnp.float32)
    size_rows = jnp.dot(onehot, size_ref[0],
                        preferred_element_type=jnp.float32)
    off_rows = jnp.dot(onehot, off_ref[0],
                       preferred_element_type=jnp.float32)
    lane2 = lax.broadcasted_iota(jnp.int32, (H, 2 * W), 1)
    sel_h = lane2 == 2 * xT
    sel_w = lane2 == 2 * xT + 1
    zf = jnp.float32(0)
    h = jnp.sum(jnp.where(sel_h, size_rows, zf), axis=1, keepdims=True)
    w = jnp.sum(jnp.where(sel_w, size_rows, zf), axis=1, keepdims=True)
    yo = jnp.sum(jnp.where(sel_h, off_rows, zf), axis=1, keepdims=True)
    xo = jnp.sum(jnp.where(sel_w, off_rows, zf), axis=1, keepdims=True)

    yf = yT.astype(jnp.float32)
    xf = xT.astype(jnp.float32)
    hh = jnp.maximum(h, 0.0)
    ww = jnp.maximum(w, 0.0)
    Hf = jnp.float32(H)
    Wf = jnp.float32(W)
    ymin = jnp.clip(yf + yo - hh / 2.0, 0.0, Hf)
    xmin = jnp.clip(xf + xo - ww / 2.0, 0.0, Wf)
    ymax = jnp.clip(yf + yo + hh / 2.0, 0.0, Hf)
    xmax = jnp.clip(xf + xo + ww / 2.0, 0.0, Wf)
    box = jnp.concatenate([ymin, xmin, ymax, xmax], axis=1)
    box = jnp.clip(box * 4.0 / 512.0, 0.0, 1.0)

    nd = jnp.sum(jnp.where((sc > 0.0) & (k_iota < K), 1, 0))
    nd_row = jnp.where(k_iota == 0, nd, 0)

    sc_out_ref[0, 0] = sc[0]
    box_out_ref[0] = box
    int_out_ref[0] = jnp.concatenate([cv, nd_row], axis=0)


def kernel(ct_heatmaps, ct_size, ct_offset):
    B, H, W, C = ct_heatmaps.shape
    K = _K
    size_r = ct_size.reshape(B, H, 2 * W)
    off_r = ct_offset.reshape(B, H, 2 * W)

    body = functools.partial(_detgen_kernel, H=H, W=W, C=C, K=K)
    sc, box, ints = pl.pallas_call(
        body,
        grid=(B,),
        in_specs=[
            pl.BlockSpec((1, H, W, C), lambda b: (b, 0, 0, 0)),
            pl.BlockSpec((1, H, 2 * W), lambda b: (b, 0, 0)),
            pl.BlockSpec((1, H, 2 * W), lambda b: (b, 0, 0)),
        ],
        out_specs=[
            pl.BlockSpec((1, 1, H), lambda b: (b, 0, 0)),
            pl.BlockSpec((1, H, 4), lambda b: (b, 0, 0)),
            pl.BlockSpec((1, 2, H), lambda b: (b, 0, 0)),
        ],
        out_shape=[
            jax.ShapeDtypeStruct((B, 1, H), jnp.float32),
            jax.ShapeDtypeStruct((B, H, 4), jnp.float32),
            jax.ShapeDtypeStruct((B, 2, H), jnp.int32),
        ],
        scratch_shapes=[pltpu.VMEM((H * W // 8, 8, C), jnp.float32)],
        compiler_params=pltpu.CompilerParams(
            dimension_semantics=("parallel",),
        ),
    )(ct_heatmaps, size_r, off_r)

    boxes = box[:, :K, :]
    channel_indices = ints[:, 0, :K]
    detection_scores = sc[:, 0, :K]
    num_detections = ints[:, 1, 0]
    return boxes, channel_indices, detection_scores, num_detections
